# baseline (speedup 1.0000x reference)
import jax
import jax.numpy as jnp
from jax.experimental import pallas as pl
from jax.experimental.pallas import tpu as pltpu


def _body(x_ref, w1_ref, nb1_ref, w2t_ref, corr_ref, o_ref):
    h = jnp.dot(x_ref[0], w1_ref[...], preferred_element_type=jnp.float32)
    z = jnp.maximum(h, nb1_ref[...])
    v = z * w2t_ref[...]
    o_ref[...] = jnp.sum(v).reshape(1, 1, 1) + corr_ref[...]


def kernel(embs, W1, b1, W2, b2):
    B, L, D = embs.shape
    H = W1.shape[1]
    w2row = W2.reshape(H)
    corr = (L * (jnp.dot(b1, w2row) + b2[0])).reshape(1, 1)

    out = pl.pallas_call(
        _body,
        grid=(B,),
        in_specs=[
            pl.BlockSpec((1, L, D), lambda i: (i, 0, 0)),
            pl.BlockSpec((D, H), lambda i: (0, 0)),
            pl.BlockSpec((1, H), lambda i: (0, 0)),
            pl.BlockSpec((1, H), lambda i: (0, 0)),
            pl.BlockSpec((1, 1), lambda i: (0, 0)),
        ],
        out_specs=pl.BlockSpec((1, 1, 1), lambda i: (i, 0, 0)),
        out_shape=jax.ShapeDtypeStruct((B, 1, 1), jnp.float32),
    )(embs, W1, (-b1).reshape(1, H), w2row.reshape(1, H), corr)
    return out.reshape(B)
